# baseline (speedup 1.0000x reference)
import functools

import jax
import jax.numpy as jnp
from jax import lax
from jax.experimental import pallas as pl
from jax.experimental.pallas import tpu as pltpu
from jax.experimental.pallas import tpu_sc as plsc

_NUM_CORES = 2
_NUM_SUBCORES = 16
_NW = _NUM_CORES * _NUM_SUBCORES
_CHUNK = 128
_NBUF = 5


def kernel(indices, table):
    B, L = indices.shape
    V, D = table.shape
    total = B * L
    per_w = total // _NW
    n_outer = per_w // (_CHUNK * _NBUF)

    idx = indices.reshape(_NW, n_outer, _NBUF, _CHUNK).astype(jnp.int32)

    mesh = plsc.VectorSubcoreMesh(core_axis_name="c", subcore_axis_name="s")

    @functools.partial(
        pl.kernel,
        out_type=jax.ShapeDtypeStruct((total, D), jnp.float32),
        mesh=mesh,
        compiler_params=pltpu.CompilerParams(use_tc_tiling_on_sc=False),
        scratch_types=[
            pltpu.VMEM((n_outer, _NBUF, _CHUNK), jnp.int32),
            pltpu.VMEM((_NBUF, _CHUNK, D), jnp.float32),
            [pltpu.SemaphoreType.DMA] * _NBUF,
            [pltpu.SemaphoreType.DMA] * _NBUF,
        ],
    )
    def gather_kernel(idx_hbm, tab_hbm, out_hbm, idx_v, rows_v, gsems, wsems):
        wid = lax.axis_index("s") * _NUM_CORES + lax.axis_index("c")
        base = wid * per_w

        pltpu.sync_copy(idx_hbm.at[wid], idx_v)

        def out_dst(g, b):
            return out_hbm.at[pl.ds(base + (g * _NBUF + b) * _CHUNK, _CHUNK)]

        def start_gather(g, b):
            pltpu.async_copy(tab_hbm.at[idx_v.at[g, b]], rows_v.at[b], gsems[b])

        def wait_gather(g, b):
            pltpu.make_async_copy(
                tab_hbm.at[idx_v.at[g, b]], rows_v.at[b], gsems[b]).wait()

        def start_write(g, b):
            pltpu.async_copy(rows_v.at[b], out_dst(g, b), wsems[b])

        def wait_write(g, b):
            pltpu.make_async_copy(rows_v.at[b], out_dst(g, b), wsems[b]).wait()

        for b in range(_NBUF):
            start_gather(0, b)

        def outer(g, carry):
            for b in range(_NBUF):
                wait_gather(g, b)
                start_write(g, b)
            for b in range(_NBUF):
                wait_write(g, b)
                start_gather(g + 1, b)
            return carry

        lax.fori_loop(0, n_outer - 1, outer, 0)

        g_last = n_outer - 1
        for b in range(_NBUF):
            wait_gather(g_last, b)
            start_write(g_last, b)
        for b in range(_NBUF):
            wait_write(g_last, b)

    out = gather_kernel(idx, table)
    return out.reshape(B, L, D)
